# baseline (speedup 1.0000x reference)
import functools

import jax
import jax.numpy as jnp
from jax import lax
from jax.experimental import pallas as pl
from jax.experimental.pallas import tpu as pltpu
from jax.experimental.pallas import tpu_sc as plsc

BATCH = 4096
SEQ_LEN = 200
EMBED_DIM = 64

NC = 2
NS = 16
NW = NC * NS

N_ROWS = BATCH * SEQ_LEN
ROWS_PER_W = N_ROWS // NW
CHUNK = 128
CHUNKS_PER_W = ROWS_PER_W // CHUNK


def _gather_kernel(x_hbm, tab_hbm, out_hbm, idx_v, rows_v, gsem):
    wid = lax.axis_index("s") * NC + lax.axis_index("c")
    pltpu.sync_copy(x_hbm.at[pl.ds(wid * CHUNKS_PER_W, CHUNKS_PER_W)], idx_v)
    row_base = wid * ROWS_PER_W

    def step(j, carry):
        pltpu.async_copy(tab_hbm.at[idx_v.at[j]], rows_v, gsem).wait()
        pltpu.sync_copy(rows_v, out_hbm.at[pl.ds(row_base + j * CHUNK, CHUNK)])
        return carry

    lax.fori_loop(0, CHUNKS_PER_W, step, 0)


def _gather(x2d, table):
    mesh = plsc.VectorSubcoreMesh(core_axis_name="c", subcore_axis_name="s")
    run = functools.partial(
        pl.kernel,
        mesh=mesh,
        compiler_params=pltpu.CompilerParams(use_tc_tiling_on_sc=False),
        out_type=jax.ShapeDtypeStruct((N_ROWS, EMBED_DIM), jnp.float32),
        scratch_types=[
            pltpu.VMEM((CHUNKS_PER_W, CHUNK), jnp.int32),
            pltpu.VMEM((CHUNK, EMBED_DIM), jnp.float32),
            pltpu.SemaphoreType.DMA,
        ],
    )(_gather_kernel)
    return run(x2d, table)


def kernel(x, table):
    x2d = x.reshape(NW * CHUNKS_PER_W, CHUNK).astype(jnp.int32)
    out = _gather(x2d, table)
    return out.reshape(BATCH, SEQ_LEN, EMBED_DIM)


# speedup vs baseline: 1.1109x; 1.1109x over previous
import functools

import jax
import jax.numpy as jnp
from jax import lax
from jax.experimental import pallas as pl
from jax.experimental.pallas import tpu as pltpu
from jax.experimental.pallas import tpu_sc as plsc

BATCH = 4096
SEQ_LEN = 200
EMBED_DIM = 64

NC = 2
NS = 16
NW = NC * NS

N_ROWS = BATCH * SEQ_LEN
ROWS_PER_W = N_ROWS // NW
CHUNK = 128
CHUNKS_PER_W = ROWS_PER_W // CHUNK


NBUF = 8
NGROUP = CHUNKS_PER_W // NBUF


def _gather_kernel(x_hbm, tab_hbm, out_hbm, idx_v, rows_v, *sems):
    gsems, ssems = sems[:NBUF], sems[NBUF:]
    wid = lax.axis_index("s") * NC + lax.axis_index("c")
    pltpu.sync_copy(x_hbm.at[pl.ds(wid * CHUNKS_PER_W, CHUNKS_PER_W)], idx_v)
    row_base = wid * ROWS_PER_W

    for b in range(NBUF):
        pltpu.async_copy(tab_hbm.at[idx_v.at[b]], rows_v.at[b], gsems[b])

    def outer(g, carry):
        for b in range(NBUF):
            j = g * NBUF + b
            dst = out_hbm.at[pl.ds(row_base + j * CHUNK, CHUNK)]
            pltpu.make_async_copy(
                tab_hbm.at[idx_v.at[j]], rows_v.at[b], gsems[b]).wait()
            pltpu.async_copy(rows_v.at[b], dst, ssems[b])

        @pl.when(g < NGROUP - 1)
        def _refill():
            for b in range(NBUF):
                j = g * NBUF + b
                dst = out_hbm.at[pl.ds(row_base + j * CHUNK, CHUNK)]
                pltpu.make_async_copy(rows_v.at[b], dst, ssems[b]).wait()
                pltpu.async_copy(
                    tab_hbm.at[idx_v.at[j + NBUF]], rows_v.at[b], gsems[b])

        return carry

    lax.fori_loop(0, NGROUP, outer, 0)

    for b in range(NBUF):
        j = (NGROUP - 1) * NBUF + b
        dst = out_hbm.at[pl.ds(row_base + j * CHUNK, CHUNK)]
        pltpu.make_async_copy(rows_v.at[b], dst, ssems[b]).wait()


def _gather(x2d, table):
    mesh = plsc.VectorSubcoreMesh(core_axis_name="c", subcore_axis_name="s")
    run = functools.partial(
        pl.kernel,
        mesh=mesh,
        compiler_params=pltpu.CompilerParams(use_tc_tiling_on_sc=False),
        out_type=jax.ShapeDtypeStruct((N_ROWS, EMBED_DIM), jnp.float32),
        scratch_types=[
            pltpu.VMEM((CHUNKS_PER_W, CHUNK), jnp.int32),
            pltpu.VMEM((NBUF, CHUNK, EMBED_DIM), jnp.float32),
        ] + [pltpu.SemaphoreType.DMA] * (2 * NBUF),
    )(_gather_kernel)
    return run(x2d, table)


def kernel(x, table):
    x2d = x.reshape(NW * CHUNKS_PER_W, CHUNK).astype(jnp.int32)
    out = _gather(x2d, table)
    return out.reshape(BATCH, SEQ_LEN, EMBED_DIM)
